# baseline (speedup 1.0000x reference)
import functools

import jax
import jax.numpy as jnp
import numpy as np
from jax import lax
from jax.experimental import pallas as pl
from jax.experimental.pallas import tpu as pltpu
from jax.experimental.pallas import tpu_sc as plsc

_PHI = (1.0 + np.sqrt(5.0)) / 2.0

NC = 2
NS = 16
NW = NC * NS
LANES = 16


def _icosahedron():
    v = np.array([[-1, _PHI, 0], [1, _PHI, 0], [-1, -_PHI, 0], [1, -_PHI, 0],
                  [0, -1, _PHI], [0, 1, _PHI], [0, -1, -_PHI], [0, 1, -_PHI],
                  [_PHI, 0, -1], [_PHI, 0, 1], [-_PHI, 0, -1], [-_PHI, 0, 1]], dtype=np.float64)
    v /= np.linalg.norm(v, axis=1, keepdims=True)
    f = np.array([[0, 11, 5], [0, 5, 1], [0, 1, 7], [0, 7, 10], [0, 10, 11],
                  [1, 5, 9], [5, 11, 4], [11, 10, 2], [10, 7, 6], [7, 1, 8],
                  [3, 9, 4], [3, 4, 2], [3, 2, 6], [3, 6, 8], [3, 8, 9],
                  [4, 9, 5], [2, 4, 11], [6, 2, 10], [8, 6, 7], [9, 8, 1]], dtype=np.int64)
    return v, f


def _geom():
    v, f = _icosahedron()
    cen = v[f].mean(axis=1)
    cen /= np.linalg.norm(cen, axis=1, keepdims=True)
    lat0 = np.arcsin(np.clip(cen[:, 2], -1.0, 1.0))
    lon0 = np.arctan2(cen[:, 1], cen[:, 0])
    vv = v[f]
    vlat = np.arcsin(np.clip(vv[:, :, 2], -1.0, 1.0))
    vlon = np.arctan2(vv[:, :, 1], vv[:, :, 0])
    dl = vlon - lon0[:, None]
    dl = (dl + np.pi) % (2 * np.pi) - np.pi
    cosc = np.sin(lat0[:, None]) * np.sin(vlat) + np.cos(lat0[:, None]) * np.cos(vlat) * np.cos(dl)
    x = np.cos(vlat) * np.sin(dl) / cosc
    y = (np.cos(lat0[:, None]) * np.sin(vlat) - np.sin(lat0[:, None]) * np.cos(vlat) * np.cos(dl)) / cosc
    ext = np.maximum(np.abs(x), np.abs(y)).max(axis=1)
    return cen, lon0, lat0, ext


def _resample_plan(h, w):
    H, Wd = 2 * h, 4 * w
    cen, lon0, lat0, ext = _geom()
    lat = np.pi / 2 - (np.arange(H) + 0.5) * np.pi / H
    lon = (np.arange(Wd) + 0.5) * 2 * np.pi / Wd - np.pi
    lon_g, lat_g = np.meshgrid(lon, lat)
    d = np.stack([np.cos(lat_g) * np.cos(lon_g), np.cos(lat_g) * np.sin(lon_g), np.sin(lat_g)], axis=-1).reshape(-1, 3)
    face = np.argmax(d @ cen.T, axis=1)
    lo, la, ex = lon0[face], lat0[face], ext[face]
    lonf, latf = lon_g.reshape(-1), lat_g.reshape(-1)
    dl = lonf - lo
    dl = (dl + np.pi) % (2 * np.pi) - np.pi
    cosc = np.sin(la) * np.sin(latf) + np.cos(la) * np.cos(latf) * np.cos(dl)
    x = np.cos(latf) * np.sin(dl) / cosc
    y = (np.cos(la) * np.sin(latf) - np.sin(la) * np.cos(latf) * np.cos(dl)) / cosc
    u = np.clip((x / ex + 1.0) * 0.5 * (w - 1), 0, w - 1)
    v = np.clip((1.0 - y / ex) * 0.5 * (h - 1), 0, h - 1)
    u0 = np.floor(u).astype(np.int64); v0 = np.floor(v).astype(np.int64)
    u1 = np.minimum(u0 + 1, w - 1); v1 = np.minimum(v0 + 1, h - 1)
    au = (u - u0).astype(np.float32); av = (v - v0).astype(np.float32)
    base = face * h * w
    idx = np.stack([base + v0 * w + u0, base + v0 * w + u1, base + v1 * w + u0, base + v1 * w + u1], axis=0)
    wts = np.stack([(1 - au) * (1 - av), au * (1 - av), (1 - au) * av, au * av], axis=0)
    return idx.astype(np.int32), wts.astype(np.float32), H, Wd


@functools.lru_cache(maxsize=None)
def _plan_arrays(h, w, chunk):
    idx, wts, H, Wd = _resample_plan(h, w)
    P = H * Wd
    ppw = P // NW
    nck = ppw // chunk
    idx_r = idx.T.reshape(NW, nck, chunk, 4).transpose(0, 1, 3, 2).copy()
    u = wts.view(np.uint32)
    wb = ((u + 0x7FFF + ((u >> 16) & 1)) >> 16).astype(np.uint32)
    wdup = ((wb << 16) | wb).view(np.float32)
    wtmp = wdup.reshape(4, NW, nck, chunk // 8, 8)
    wts_r = np.empty((NW, nck, chunk // 8, 2, 16), np.float32)
    wts_r[..., 0, :8] = wtmp[0]
    wts_r[..., 0, 8:] = wtmp[1]
    wts_r[..., 1, :8] = wtmp[2]
    wts_r[..., 1, 8:] = wtmp[3]
    return idx_r, wts_r, H, Wd


def _sc_remap(table, idx_r, wts_r, P, chunk):
    D2 = table.shape[1]
    nck = idx_r.shape[1]
    ppw = nck * chunk
    mesh = plsc.VectorSubcoreMesh(core_axis_name="c", subcore_axis_name="s",
                                  num_cores=NC, num_subcores=NS)

    def body(table_hbm, idx_hbm, wts_hbm, out_hbm, idx_v, wts_v, rows_v, out_v,
             gsem, osem):
        wid = lax.axis_index("s") * NC + lax.axis_index("c")
        base = wid * ppw
        pltpu.sync_copy(idx_hbm.at[wid], idx_v)
        pltpu.sync_copy(wts_hbm.at[wid], wts_v)

        def issue_gathers(c, p):
            for k in range(4):
                pltpu.async_copy(table_hbm.at[idx_v.at[c, k]],
                                 rows_v.at[p, k], gsem.at[p])

        def wait_gathers(c, p):
            for k in range(4):
                pltpu.make_async_copy(table_hbm.at[idx_v.at[c, k]],
                                      rows_v.at[p, k], gsem.at[p]).wait()

        def compute_chunk(c, p):
            def oct_body(q, _):
                wq0 = wts_v[c, q, 0, :]
                wq1 = wts_v[c, q, 1, :]
                for i2 in range(8):
                    j = q * 8 + i2
                    w0 = plsc.bitcast(jnp.full((LANES,), wq0[i2]), jnp.bfloat16)
                    w1 = plsc.bitcast(jnp.full((LANES,), wq0[8 + i2]), jnp.bfloat16)
                    w2 = plsc.bitcast(jnp.full((LANES,), wq1[i2]), jnp.bfloat16)
                    w3 = plsc.bitcast(jnp.full((LANES,), wq1[8 + i2]), jnp.bfloat16)
                    for v in range(D2 // LANES):
                        sl = pl.ds(v * LANES, LANES)
                        r0 = plsc.bitcast(rows_v[p, 0, j, sl], jnp.bfloat16)
                        r1 = plsc.bitcast(rows_v[p, 1, j, sl], jnp.bfloat16)
                        r2 = plsc.bitcast(rows_v[p, 2, j, sl], jnp.bfloat16)
                        r3 = plsc.bitcast(rows_v[p, 3, j, sl], jnp.bfloat16)
                        s = r0 * w0 + r1 * w1 + r2 * w2 + r3 * w3
                        out_v[p, j, sl] = plsc.bitcast(s, jnp.float32)
                return 0

            lax.fori_loop(0, chunk // 8, oct_body, 0)

        def drain_out(c, p):
            pltpu.make_async_copy(out_v.at[p],
                                  out_hbm.at[pl.ds(base + c * chunk, chunk)],
                                  osem.at[p]).wait()

        def write_out(c, p):
            pltpu.async_copy(out_v.at[p],
                             out_hbm.at[pl.ds(base + c * chunk, chunk)],
                             osem.at[p])

        issue_gathers(0, 0)

        def pair_body(i, _):
            c0 = 2 * i
            c1 = c0 + 1

            issue_gathers(c1, 1)
            wait_gathers(c0, 0)

            @pl.when(i >= 1)
            def _():
                drain_out(c0 - 2, 0)

            compute_chunk(c0, 0)
            write_out(c0, 0)

            @pl.when(c0 + 2 < nck)
            def _():
                issue_gathers(c0 + 2, 0)

            wait_gathers(c1, 1)

            @pl.when(i >= 1)
            def _():
                drain_out(c1 - 2, 1)

            compute_chunk(c1, 1)
            write_out(c1, 1)
            return 0

        lax.fori_loop(0, nck // 2, pair_body, 0)

        drain_out(nck - 2, 0)
        drain_out(nck - 1, 1)

    run = pl.kernel(
        body,
        out_type=jax.ShapeDtypeStruct((P, D2), jnp.float32),
        mesh=mesh,
        compiler_params=pltpu.CompilerParams(needs_layout_passes=False),
        scratch_types=[
            pltpu.VMEM((nck, 4, chunk), jnp.int32),
            pltpu.VMEM((nck, chunk // 8, 2, LANES), jnp.float32),
            pltpu.VMEM((2, 4, chunk, D2), jnp.float32),
            pltpu.VMEM((2, chunk, D2), jnp.float32),
            pltpu.SemaphoreType.DMA((2,)),
            pltpu.SemaphoreType.DMA((2,)),
        ],
    )
    return run(table, jnp.asarray(idx_r), jnp.asarray(wts_r))


def _tc_pack(tan, blk=1024):
    n, b, c, h, w = tan.shape
    D = b * c
    hw = h * w
    x3 = tan.reshape(n, D, hw)

    def body(tan_ref, out_ref):
        x = tan_ref[0]
        bot = x[:D // 2, :].astype(jnp.bfloat16)
        top = x[D // 2:, :].astype(jnp.bfloat16)
        ub = lax.bitcast_convert_type(bot, jnp.uint16).astype(jnp.uint32)
        ut = lax.bitcast_convert_type(top, jnp.uint16).astype(jnp.uint32)
        words = (ut << 16) | ub
        out_ref[...] = lax.bitcast_convert_type(words.T, jnp.float32)

    return pl.pallas_call(
        body,
        grid=(n, hw // blk),
        in_specs=[pl.BlockSpec((1, D, blk), lambda i, j: (i, 0, j))],
        out_specs=pl.BlockSpec((blk, D // 2), lambda i, j: (i * (hw // blk) + j, 0)),
        out_shape=jax.ShapeDtypeStruct((n * hw, D // 2), jnp.float32),
    )(x3)


def _tc_unpack(pk, D, blk=512):
    P = pk.shape[0]

    def body(pk_ref, out_ref):
        u = lax.bitcast_convert_type(pk_ref[...], jnp.uint32).T
        lo = lax.bitcast_convert_type(u << 16, jnp.float32)
        hi = lax.bitcast_convert_type(u & jnp.uint32(0xFFFF0000), jnp.float32)
        out_ref[:D // 2, :] = lo
        out_ref[D // 2:, :] = hi

    return pl.pallas_call(
        body,
        grid=(P // blk,),
        in_specs=[pl.BlockSpec((blk, D // 2), lambda i: (i, 0))],
        out_specs=pl.BlockSpec((D, blk), lambda i: (0, i)),
        out_shape=jax.ShapeDtypeStruct((D, P), jnp.float32),
    )(pk)


@jax.jit
def kernel(tan):
    n, b, c, h, w = tan.shape
    chunk = 32
    idx_r, wts_r, H, Wd = _plan_arrays(h, w, chunk)
    P = H * Wd
    D = b * c
    table = _tc_pack(tan)
    out = _sc_remap(table, idx_r, wts_r, P, chunk)
    return _tc_unpack(out, D).reshape(b, c, H, Wd)


# speedup vs baseline: 1.2146x; 1.2146x over previous
import functools

import jax
import jax.numpy as jnp
import numpy as np
from jax import lax
from jax.experimental import pallas as pl
from jax.experimental.pallas import tpu as pltpu
from jax.experimental.pallas import tpu_sc as plsc

_PHI = (1.0 + np.sqrt(5.0)) / 2.0

NC = 2
NS = 16
NW = NC * NS
LANES = 16


def _icosahedron():
    v = np.array([[-1, _PHI, 0], [1, _PHI, 0], [-1, -_PHI, 0], [1, -_PHI, 0],
                  [0, -1, _PHI], [0, 1, _PHI], [0, -1, -_PHI], [0, 1, -_PHI],
                  [_PHI, 0, -1], [_PHI, 0, 1], [-_PHI, 0, -1], [-_PHI, 0, 1]], dtype=np.float64)
    v /= np.linalg.norm(v, axis=1, keepdims=True)
    f = np.array([[0, 11, 5], [0, 5, 1], [0, 1, 7], [0, 7, 10], [0, 10, 11],
                  [1, 5, 9], [5, 11, 4], [11, 10, 2], [10, 7, 6], [7, 1, 8],
                  [3, 9, 4], [3, 4, 2], [3, 2, 6], [3, 6, 8], [3, 8, 9],
                  [4, 9, 5], [2, 4, 11], [6, 2, 10], [8, 6, 7], [9, 8, 1]], dtype=np.int64)
    return v, f


def _geom():
    v, f = _icosahedron()
    cen = v[f].mean(axis=1)
    cen /= np.linalg.norm(cen, axis=1, keepdims=True)
    lat0 = np.arcsin(np.clip(cen[:, 2], -1.0, 1.0))
    lon0 = np.arctan2(cen[:, 1], cen[:, 0])
    vv = v[f]
    vlat = np.arcsin(np.clip(vv[:, :, 2], -1.0, 1.0))
    vlon = np.arctan2(vv[:, :, 1], vv[:, :, 0])
    dl = vlon - lon0[:, None]
    dl = (dl + np.pi) % (2 * np.pi) - np.pi
    cosc = np.sin(lat0[:, None]) * np.sin(vlat) + np.cos(lat0[:, None]) * np.cos(vlat) * np.cos(dl)
    x = np.cos(vlat) * np.sin(dl) / cosc
    y = (np.cos(lat0[:, None]) * np.sin(vlat) - np.sin(lat0[:, None]) * np.cos(vlat) * np.cos(dl)) / cosc
    ext = np.maximum(np.abs(x), np.abs(y)).max(axis=1)
    return cen, lon0, lat0, ext


def _resample_plan(h, w):
    H, Wd = 2 * h, 4 * w
    cen, lon0, lat0, ext = _geom()
    lat = np.pi / 2 - (np.arange(H) + 0.5) * np.pi / H
    lon = (np.arange(Wd) + 0.5) * 2 * np.pi / Wd - np.pi
    lon_g, lat_g = np.meshgrid(lon, lat)
    d = np.stack([np.cos(lat_g) * np.cos(lon_g), np.cos(lat_g) * np.sin(lon_g), np.sin(lat_g)], axis=-1).reshape(-1, 3)
    face = np.argmax(d @ cen.T, axis=1)
    lo, la, ex = lon0[face], lat0[face], ext[face]
    lonf, latf = lon_g.reshape(-1), lat_g.reshape(-1)
    dl = lonf - lo
    dl = (dl + np.pi) % (2 * np.pi) - np.pi
    cosc = np.sin(la) * np.sin(latf) + np.cos(la) * np.cos(latf) * np.cos(dl)
    x = np.cos(latf) * np.sin(dl) / cosc
    y = (np.cos(la) * np.sin(latf) - np.sin(la) * np.cos(latf) * np.cos(dl)) / cosc
    u = np.clip((x / ex + 1.0) * 0.5 * (w - 1), 0, w - 1)
    v = np.clip((1.0 - y / ex) * 0.5 * (h - 1), 0, h - 1)
    u0 = np.floor(u).astype(np.int64); v0 = np.floor(v).astype(np.int64)
    u1 = np.minimum(u0 + 1, w - 1); v1 = np.minimum(v0 + 1, h - 1)
    au = (u - u0).astype(np.float32); av = (v - v0).astype(np.float32)
    base = face * h * w
    idx = np.stack([base + v0 * w + u0, base + v0 * w + u1, base + v1 * w + u0, base + v1 * w + u1], axis=0)
    wts = np.stack([(1 - au) * (1 - av), au * (1 - av), (1 - au) * av, au * av], axis=0)
    return idx.astype(np.int32), wts.astype(np.float32), H, Wd


@functools.lru_cache(maxsize=None)
def _plan_arrays(h, w, chunk):
    idx, wts, H, Wd = _resample_plan(h, w)
    P = H * Wd
    ppw = P // NW
    nck = ppw // chunk
    idx_r = idx.T.reshape(NW, nck, chunk, 4).transpose(0, 1, 3, 2).copy()
    u = wts.view(np.uint32)
    wb = ((u + 0x7FFF + ((u >> 16) & 1)) >> 16).astype(np.uint32)
    wdup = ((wb << 16) | wb).view(np.float32)
    wtmp = wdup.reshape(4, NW, nck, chunk // 8, 8)
    wts_r = np.empty((NW, nck, chunk // 8, 2, 16), np.float32)
    wts_r[..., 0, :8] = wtmp[0]
    wts_r[..., 0, 8:] = wtmp[1]
    wts_r[..., 1, :8] = wtmp[2]
    wts_r[..., 1, 8:] = wtmp[3]
    return idx_r, wts_r, H, Wd


def _sc_remap(table, idx_r, wts_r, P, chunk):
    D2 = table.shape[1]
    nck = idx_r.shape[1]
    ppw = nck * chunk
    mesh = plsc.VectorSubcoreMesh(core_axis_name="c", subcore_axis_name="s",
                                  num_cores=NC, num_subcores=NS)

    def body(table_hbm, idx_hbm, wts_hbm, out_hbm, idx_v, wts_v, rows_v, out_v,
             gsem, osem):
        wid = lax.axis_index("s") * NC + lax.axis_index("c")
        base = wid * ppw
        pltpu.sync_copy(idx_hbm.at[wid], idx_v)
        pltpu.sync_copy(wts_hbm.at[wid], wts_v)

        def issue_gathers(c, p):
            for k in range(4):
                pltpu.async_copy(table_hbm.at[idx_v.at[c, k]],
                                 rows_v.at[p, k], gsem.at[p])

        def wait_gathers(c, p):
            for k in range(4):
                pltpu.make_async_copy(table_hbm.at[idx_v.at[c, k]],
                                      rows_v.at[p, k], gsem.at[p]).wait()

        def compute_chunk(c, p):
            def oct_body(q, _):
                wq0 = wts_v[c, q, 0, :]
                wq1 = wts_v[c, q, 1, :]
                for i2 in range(8):
                    j = q * 8 + i2
                    w0 = plsc.bitcast(jnp.full((LANES,), wq0[i2]), jnp.bfloat16)
                    w1 = plsc.bitcast(jnp.full((LANES,), wq0[8 + i2]), jnp.bfloat16)
                    w2 = plsc.bitcast(jnp.full((LANES,), wq1[i2]), jnp.bfloat16)
                    w3 = plsc.bitcast(jnp.full((LANES,), wq1[8 + i2]), jnp.bfloat16)
                    for v in range(D2 // LANES):
                        sl = pl.ds(v * LANES, LANES)
                        r0 = plsc.bitcast(rows_v[p, 0, j, sl], jnp.bfloat16)
                        r1 = plsc.bitcast(rows_v[p, 1, j, sl], jnp.bfloat16)
                        r2 = plsc.bitcast(rows_v[p, 2, j, sl], jnp.bfloat16)
                        r3 = plsc.bitcast(rows_v[p, 3, j, sl], jnp.bfloat16)
                        s = r0 * w0 + r1 * w1 + r2 * w2 + r3 * w3
                        out_v[p, j, sl] = plsc.bitcast(s, jnp.float32)
                return 0

            lax.fori_loop(0, chunk // 8, oct_body, 0)

        def drain_out(c, p):
            pltpu.make_async_copy(out_v.at[p],
                                  out_hbm.at[pl.ds(base + c * chunk, chunk)],
                                  osem.at[p]).wait()

        def write_out(c, p):
            pltpu.async_copy(out_v.at[p],
                             out_hbm.at[pl.ds(base + c * chunk, chunk)],
                             osem.at[p])

        issue_gathers(0, 0)

        def pair_body(i, _):
            c0 = 2 * i
            c1 = c0 + 1

            issue_gathers(c1, 1)
            wait_gathers(c0, 0)

            @pl.when(i >= 1)
            def _():
                drain_out(c0 - 2, 0)

            compute_chunk(c0, 0)
            write_out(c0, 0)

            @pl.when(c0 + 2 < nck)
            def _():
                issue_gathers(c0 + 2, 0)

            wait_gathers(c1, 1)

            @pl.when(i >= 1)
            def _():
                drain_out(c1 - 2, 1)

            compute_chunk(c1, 1)
            write_out(c1, 1)
            return 0

        lax.fori_loop(0, nck // 2, pair_body, 0)

        drain_out(nck - 2, 0)
        drain_out(nck - 1, 1)

    run = pl.kernel(
        body,
        out_type=jax.ShapeDtypeStruct((P, D2), jnp.float32),
        mesh=mesh,
        compiler_params=pltpu.CompilerParams(needs_layout_passes=False),
        scratch_types=[
            pltpu.VMEM((nck, 4, chunk), jnp.int32),
            pltpu.VMEM((nck, chunk // 8, 2, LANES), jnp.float32),
            pltpu.VMEM((2, 4, chunk, D2), jnp.float32),
            pltpu.VMEM((2, chunk, D2), jnp.float32),
            pltpu.SemaphoreType.DMA((2,)),
            pltpu.SemaphoreType.DMA((2,)),
        ],
    )
    return run(table, jnp.asarray(idx_r), jnp.asarray(wts_r))


def _tc_pack(table_f32, blk=2048):
    R, D = table_f32.shape

    def body(x_ref, out_ref):
        x = x_ref[...]
        lo = x[:, :D // 2].astype(jnp.bfloat16)
        hi = x[:, D // 2:].astype(jnp.bfloat16)
        ul = lax.bitcast_convert_type(lo, jnp.uint16).astype(jnp.uint32)
        uh = lax.bitcast_convert_type(hi, jnp.uint16).astype(jnp.uint32)
        out_ref[...] = lax.bitcast_convert_type((uh << 16) | ul, jnp.float32)

    return pl.pallas_call(
        body,
        grid=(R // blk,),
        in_specs=[pl.BlockSpec((blk, D), lambda i: (i, 0))],
        out_specs=pl.BlockSpec((blk, D // 2), lambda i: (i, 0)),
        out_shape=jax.ShapeDtypeStruct((R, D // 2), jnp.float32),
    )(table_f32)


def _tc_unpack(pk, D, blk=512):
    P = pk.shape[0]

    def body(pk_ref, out_ref):
        u = lax.bitcast_convert_type(pk_ref[...], jnp.uint32).T
        lo = lax.bitcast_convert_type(u << 16, jnp.float32)
        hi = lax.bitcast_convert_type(u & jnp.uint32(0xFFFF0000), jnp.float32)
        out_ref[:D // 2, :] = lo
        out_ref[D // 2:, :] = hi

    return pl.pallas_call(
        body,
        grid=(P // blk,),
        in_specs=[pl.BlockSpec((blk, D // 2), lambda i: (i, 0))],
        out_specs=pl.BlockSpec((D, blk), lambda i: (0, i)),
        out_shape=jax.ShapeDtypeStruct((D, P), jnp.float32),
    )(pk)


@jax.jit
def kernel(tan):
    n, b, c, h, w = tan.shape
    chunk = 32
    idx_r, wts_r, H, Wd = _plan_arrays(h, w, chunk)
    P = H * Wd
    D = b * c
    table_f32 = jnp.transpose(tan.astype(jnp.float32), (0, 3, 4, 1, 2)).reshape(n * h * w, D)
    table = _tc_pack(table_f32)
    out = _sc_remap(table, idx_r, wts_r, P, chunk)
    return _tc_unpack(out, D).reshape(b, c, H, Wd)
